# baseline (speedup 1.0000x reference)
import functools

import jax
import jax.numpy as jnp
from jax import lax
from jax.experimental import pallas as pl
from jax.experimental.pallas import tpu as pltpu
from jax.experimental.pallas import tpu_sc as plsc

H = 768
E = 8
INTER = 2048
T = 2048
TP = 2 * T
TM = 256
NT = 23
NPAD = NT * TM
NW = 32
GPW = NPAD // NW
CPW = T // NW

_DEFAULT = jax.lax.Precision.DEFAULT


def _router_body(x_ref, wg_ref, idx_ref, w_ref):
    logits = jnp.dot(x_ref[...], wg_ref[...], preferred_element_type=jnp.float32)
    colid = jax.lax.broadcasted_iota(jnp.int32, logits.shape, 1)
    m1 = jnp.max(logits, axis=1, keepdims=True)
    idx1 = jnp.min(jnp.where(logits == m1, colid, E), axis=1, keepdims=True)
    l2 = jnp.where(colid == idx1, -jnp.inf, logits)
    m2 = jnp.max(l2, axis=1, keepdims=True)
    idx2 = jnp.min(jnp.where(l2 == m2, colid, E), axis=1, keepdims=True)
    t = jnp.exp(m2 - m1)
    w_top = 1.0 / (1.0 + t)
    w_sec = t / (1.0 + t)
    idx_ref[...] = jnp.concatenate([idx1, idx2], axis=1)
    w_ref[...] = jnp.concatenate([w_top, w_sec], axis=1)


def _router(x2d, Wg):
    return pl.pallas_call(
        _router_body,
        out_shape=[
            jax.ShapeDtypeStruct((T, 2), jnp.int32),
            jax.ShapeDtypeStruct((T, 2), jnp.float32),
        ],
    )(x2d, Wg)


_sc_mesh = plsc.VectorSubcoreMesh(core_axis_name="c", subcore_axis_name="s")

_GA, _GB = 96, 88


@functools.partial(
    pl.kernel, mesh=_sc_mesh,
    out_type=jax.ShapeDtypeStruct((NPAD, H), jnp.float32),
    scratch_types=[
        pltpu.VMEM((_GA,), jnp.int32),
        pltpu.VMEM((_GB,), jnp.int32),
        pltpu.VMEM((_GA, H), jnp.float32),
        pltpu.SemaphoreType.DMA,
    ],
)
def _sc_gather(x_hbm, idx_hbm, xs_hbm, idx_a, idx_b, rows_v, sem):
    wid = lax.axis_index("s") * 2 + lax.axis_index("c")
    base = wid * GPW
    pltpu.sync_copy(idx_hbm.at[pl.ds(base, _GA)], idx_a)
    pltpu.sync_copy(idx_hbm.at[pl.ds(base + _GA, _GB)], idx_b)
    pltpu.async_copy(x_hbm.at[idx_a], rows_v, sem).wait()
    pltpu.sync_copy(rows_v, xs_hbm.at[pl.ds(base, _GA)])
    pltpu.async_copy(x_hbm.at[idx_b], rows_v.at[pl.ds(0, _GB)], sem).wait()
    pltpu.sync_copy(rows_v.at[pl.ds(0, _GB)], xs_hbm.at[pl.ds(base + _GA, _GB)])


def _group_body(sched_ref, xs_ref, w_ref, w1_ref, w3_ref, w2_ref, ys_ref):
    xt = xs_ref[...]
    a = jnp.dot(xt, w1_ref[0], preferred_element_type=jnp.float32,
                precision=_DEFAULT)
    b = jnp.dot(xt, w3_ref[0], preferred_element_type=jnp.float32,
                precision=_DEFAULT)
    h = (a * jax.nn.sigmoid(a)) * b * w_ref[...]
    ys_ref[...] = jnp.dot(h, w2_ref[0], preferred_element_type=jnp.float32,
                          precision=_DEFAULT)


@jax.jit
def _grouped(sched, xs, w_pad, W1, W3, W2):
    grid_spec = pltpu.PrefetchScalarGridSpec(
        num_scalar_prefetch=1,
        grid=(NT,),
        in_specs=[
            pl.BlockSpec((TM, H), lambda s, sched: (s, 0)),
            pl.BlockSpec((TM, 1), lambda s, sched: (s, 0)),
            pl.BlockSpec((1, H, INTER), lambda s, sched: (sched[s], 0, 0)),
            pl.BlockSpec((1, H, INTER), lambda s, sched: (sched[s], 0, 0)),
            pl.BlockSpec((1, INTER, H), lambda s, sched: (sched[s], 0, 0)),
        ],
        out_specs=pl.BlockSpec((TM, H), lambda s, sched: (s, 0)),
    )
    return pl.pallas_call(
        _group_body,
        grid_spec=grid_spec,
        out_shape=jax.ShapeDtypeStruct((NPAD, H), jnp.float32),
        compiler_params=pltpu.CompilerParams(
            dimension_semantics=("arbitrary",),
        ),
    )(sched, xs, w_pad, W1, W3, W2)


@functools.partial(
    pl.kernel, mesh=_sc_mesh,
    out_type=jax.ShapeDtypeStruct((T, H), jnp.float32),
    scratch_types=[
        pltpu.VMEM((2 * (CPW // 2),), jnp.int32),
        pltpu.VMEM((2 * (CPW // 2), H), jnp.float32),
        pltpu.VMEM((CPW // 2, H), jnp.float32),
        pltpu.SemaphoreType.DMA,
    ],
)
def _sc_combine(ys_hbm, pos_hbm, out_hbm, pidx, rows_v, out_v, sem):
    wid = lax.axis_index("s") * 2 + lax.axis_index("c")
    half = CPW // 2

    for c in range(2):
        pbase = wid * 2 * CPW + c * 2 * half
        pltpu.sync_copy(pos_hbm.at[pl.ds(pbase, 2 * half)], pidx)
        pltpu.async_copy(ys_hbm.at[pidx], rows_v, sem).wait()

        def body(i, carry):
            for j in range(H // 16):
                s = 16 * j
                out_v[i, pl.ds(s, 16)] = (
                    rows_v[2 * i, pl.ds(s, 16)] + rows_v[2 * i + 1, pl.ds(s, 16)]
                )
            return carry

        lax.fori_loop(0, half, body, 0)
        pltpu.sync_copy(out_v, out_hbm.at[pl.ds(wid * CPW + c * half, half)])


@jax.jit
def _moe(x2d, Wg, W1, W3, W2):
    idx12, w12 = _router(x2d, Wg)
    ids = idx12.reshape(TP)
    pw = w12.reshape(TP)
    onehot = (ids[:, None] == jnp.arange(E, dtype=jnp.int32)[None, :]).astype(
        jnp.int32)
    inc = jnp.cumsum(onehot, axis=0)
    rank = jnp.take_along_axis(inc, ids[:, None], axis=1)[:, 0] - 1
    counts = inc[-1]
    tiles = (counts + TM - 1) // TM
    tile_cum = jnp.cumsum(tiles)
    row_start = (tile_cum - tiles) * TM
    ppos = (row_start[ids] + rank).astype(jnp.int32)
    sched = jnp.minimum(
        (jnp.arange(NT, dtype=jnp.int32)[:, None] >= tile_cum[None, :]).sum(
            axis=1), E - 1).astype(jnp.int32)
    tok = (jnp.arange(TP, dtype=jnp.int32) // 2).astype(jnp.int32)
    ids_pad = jnp.zeros((NPAD,), jnp.int32).at[ppos].set(
        tok, mode="drop", unique_indices=True)
    w_pad = jnp.zeros((NPAD,), jnp.float32).at[ppos].set(
        pw, mode="drop", unique_indices=True)

    xs = _sc_gather(x2d, ids_pad)
    ys = _grouped(sched, xs, w_pad.reshape(NPAD, 1), W1, W3, W2)
    return _sc_combine(ys, ppos)


def kernel(x, Wg, W1, W3, W2):
    B, S, Hd = x.shape
    out = _moe(x.reshape(-1, Hd), Wg, W1, W3, W2)
    return out.reshape(B, S, Hd)
